# Initial kernel scaffold; baseline (speedup 1.0000x reference)
import jax
import jax.numpy as jnp
from jax.experimental import pallas as pl


def kernel(token_ids, ME):
    raise NotImplementedError("write your pallas kernel here")


# speedup vs baseline: 1.8317x; 1.8317x over previous
import functools

import jax
import jax.numpy as jnp
from jax import lax
from jax.experimental import pallas as pl
from jax.experimental.pallas import tpu as pltpu
from jax.experimental.pallas import tpu_sc as plsc

NUM_CORES = 2
NUM_SUBCORES = 16
NUM_WORKERS = NUM_CORES * NUM_SUBCORES

CHUNK = 128
GATHERS_PER_BUF = 4
BUF_ROWS = CHUNK * GATHERS_PER_BUF


def _emb_kernel(B, D, b_per_w, n_chunks, n_groups):
    mesh = plsc.VectorSubcoreMesh(core_axis_name="c", subcore_axis_name="s")

    @functools.partial(
        pl.kernel,
        out_type=jax.ShapeDtypeStruct((B, D), jnp.float32),
        mesh=mesh,
        scratch_types=[
            pltpu.VMEM((n_chunks, CHUNK), jnp.int32),
            pltpu.VMEM((BUF_ROWS, D), jnp.float32),
            pltpu.SemaphoreType.DMA,
        ],
        compiler_params=pltpu.CompilerParams(use_tc_tiling_on_sc=False),
    )
    def emb(idx_hbm, table_hbm, out_hbm, idx_v, rows_v, gsem):
        wid = lax.axis_index("s") * NUM_CORES + lax.axis_index("c")
        base = wid * b_per_w
        pltpu.sync_copy(idx_hbm.at[wid], idx_v)

        @pl.loop(0, n_groups)
        def _group(g):
            descs = []
            for j in range(GATHERS_PER_BUF):
                descs.append(pltpu.async_copy(
                    table_hbm.at[idx_v.at[g * GATHERS_PER_BUF + j]],
                    rows_v.at[pl.ds(j * CHUNK, CHUNK)],
                    gsem,
                ))
            for d in descs:
                d.wait()
            pltpu.sync_copy(
                rows_v,
                out_hbm.at[pl.ds(base + g * BUF_ROWS, BUF_ROWS)],
            )

    return emb


def kernel(token_ids, ME):
    B0, S = token_ids.shape
    V, D = ME.shape
    B = B0 * S
    b_per_w = B // NUM_WORKERS
    n_chunks = b_per_w // CHUNK
    n_groups = b_per_w // BUF_ROWS
    idx = token_ids.reshape(NUM_WORKERS, n_chunks, CHUNK)
    out = _emb_kernel(B, D, b_per_w, n_chunks, n_groups)(idx, ME)
    return out.reshape(B0, S, D)


# speedup vs baseline: 1.8729x; 1.0225x over previous
import functools

import jax
import jax.numpy as jnp
from jax import lax
from jax.experimental import pallas as pl
from jax.experimental.pallas import tpu as pltpu
from jax.experimental.pallas import tpu_sc as plsc

NUM_CORES = 2
NUM_SUBCORES = 16
NUM_WORKERS = NUM_CORES * NUM_SUBCORES

CHUNK = 128
GATHERS_PER_BUF = 2
BUF_ROWS = CHUNK * GATHERS_PER_BUF
NBUF = 4
KAHEAD = 2


def _emb_kernel(B, D, b_per_w, n_chunks, n_groups):
    mesh = plsc.VectorSubcoreMesh(core_axis_name="c", subcore_axis_name="s")

    @functools.partial(
        pl.kernel,
        out_type=jax.ShapeDtypeStruct((B, D), jnp.float32),
        mesh=mesh,
        scratch_types=[
            pltpu.VMEM((n_chunks, CHUNK), jnp.int32),
            pltpu.VMEM((NBUF, BUF_ROWS, D), jnp.float32),
        ] + [pltpu.SemaphoreType.DMA] * (2 * NBUF),
        compiler_params=pltpu.CompilerParams(use_tc_tiling_on_sc=False),
    )
    def emb(idx_hbm, table_hbm, out_hbm, idx_v, rows_v, *sems):
        gsems, osems = sems[:NBUF], sems[NBUF:]
        wid = lax.axis_index("s") * NUM_CORES + lax.axis_index("c")
        base = wid * b_per_w
        pltpu.sync_copy(idx_hbm.at[wid], idx_v)

        def fire(g, b):
            for j in range(GATHERS_PER_BUF):
                pltpu.async_copy(
                    table_hbm.at[idx_v.at[g * GATHERS_PER_BUF + j]],
                    rows_v.at[b, pl.ds(j * CHUNK, CHUNK)],
                    gsems[b],
                )

        def drain_gather(b):
            for j in range(GATHERS_PER_BUF):
                pltpu.make_async_copy(
                    table_hbm.at[idx_v.at[j]],
                    rows_v.at[b, pl.ds(j * CHUNK, CHUNK)],
                    gsems[b],
                ).wait()

        def start_out(g, b):
            pltpu.async_copy(
                rows_v.at[b],
                out_hbm.at[pl.ds(base + g * BUF_ROWS, BUF_ROWS)],
                osems[b],
            )

        def wait_out(b):
            pltpu.make_async_copy(
                rows_v.at[b],
                out_hbm.at[pl.ds(base, BUF_ROWS)],
                osems[b],
            ).wait()

        def visit(g, b, bk, do_fire, do_owait):
            if do_fire:
                if do_owait:
                    wait_out(bk)
                fire(g + KAHEAD, bk)
            drain_gather(b)
            start_out(g, b)

        for g in range(KAHEAD):
            fire(g, g % NBUF)
        for g in range(NBUF - KAHEAD):
            visit(g, g % NBUF, (g + KAHEAD) % NBUF, True, False)
        lo, hi = NBUF - KAHEAD, n_groups - KAHEAD
        assert (hi - lo) % NBUF == 0

        @pl.loop(lo, hi, step=NBUF)
        def _steady(t):
            for i in range(NBUF):
                b = (lo + i) % NBUF
                visit(t + i, b, (b + KAHEAD) % NBUF, True, True)

        for g in range(n_groups - KAHEAD, n_groups):
            visit(g, g % NBUF, 0, False, False)
        for b in range(NBUF):
            wait_out(b)

    return emb


def kernel(token_ids, ME):
    B0, S = token_ids.shape
    V, D = ME.shape
    B = B0 * S
    b_per_w = B // NUM_WORKERS
    n_chunks = b_per_w // CHUNK
    n_groups = b_per_w // BUF_ROWS
    idx = token_ids.reshape(NUM_WORKERS, n_chunks, CHUNK)
    out = _emb_kernel(B, D, b_per_w, n_chunks, n_groups)(idx, ME)
    return out.reshape(B0, S, D)
